# Initial kernel scaffold; baseline (speedup 1.0000x reference)
import jax
import jax.numpy as jnp
from jax.experimental import pallas as pl


def kernel(fnums, emb_table, pos_table):
    raise NotImplementedError("write your pallas kernel here")


# speedup vs baseline: 1.0167x; 1.0167x over previous
import functools

import jax
import jax.numpy as jnp
from jax import lax
from jax.experimental import pallas as pl
from jax.experimental.pallas import tpu as pltpu
from jax.experimental.pallas import tpu_sc as plsc

_INFO = plsc.get_sparse_core_info()
_NC, _NS = _INFO.num_cores, _INFO.num_subcores
_NW = _NC * _NS

_SEQ = 200
_D = 128
_BPW = 8
_NACT = _SEQ // _BPW

_mesh = plsc.VectorSubcoreMesh(core_axis_name="c", subcore_axis_name="s")


@functools.partial(
    pl.kernel,
    mesh=_mesh,
    out_type=jax.ShapeDtypeStruct((_SEQ, 2 * _D), jnp.float32),
    scratch_types=[
        pltpu.VMEM((_BPW,), jnp.int32),
        pltpu.VMEM((_BPW, _D), jnp.float32),
        pltpu.VMEM((_BPW, _D), jnp.float32),
        pltpu.SemaphoreType.DMA,
    ],
)
def _encode(idx_hbm, emb_hbm, pos_hbm, out_hbm, idx_v, emb_v, pos_v, sem):
    wid = lax.axis_index("s") * _NC + lax.axis_index("c")

    @pl.when(wid < _NACT)
    def _():
        base = wid * _BPW
        pltpu.sync_copy(idx_hbm.at[pl.ds(base, _BPW)], idx_v)
        gat = pltpu.async_copy(emb_hbm.at[idx_v], emb_v, sem)
        pltpu.sync_copy(pos_hbm.at[pl.ds(base, _BPW)], pos_v)
        pltpu.sync_copy(pos_v, out_hbm.at[pl.ds(base, _BPW), pl.ds(_D, _D)])
        gat.wait()
        pltpu.sync_copy(emb_v, out_hbm.at[pl.ds(base, _BPW), pl.ds(0, _D)])


def kernel(fnums, emb_table, pos_table):
    idx = fnums.astype(jnp.int32)
    return _encode(idx, emb_table, pos_table)


# speedup vs baseline: 1.0309x; 1.0140x over previous
import functools

import jax
import jax.numpy as jnp
from jax import lax
from jax.experimental import pallas as pl
from jax.experimental.pallas import tpu as pltpu
from jax.experimental.pallas import tpu_sc as plsc

_INFO = plsc.get_sparse_core_info()
_NC, _NS = _INFO.num_cores, _INFO.num_subcores
_NW = _NC * _NS

_SEQ = 200
_D = 128
_BPW = 8
_NACT = _SEQ // _BPW

_mesh = plsc.VectorSubcoreMesh(core_axis_name="c", subcore_axis_name="s")


@functools.partial(
    pl.kernel,
    mesh=_mesh,
    out_type=jax.ShapeDtypeStruct((_SEQ, 2 * _D), jnp.float32),
    scratch_types=[
        pltpu.VMEM((_BPW,), jnp.int32),
        pltpu.VMEM((_BPW, 2 * _D), jnp.float32),
        pltpu.SemaphoreType.DMA,
        pltpu.SemaphoreType.DMA,
        pltpu.SemaphoreType.DMA,
    ],
)
def _encode(idx_hbm, emb_hbm, pos_hbm, out_hbm, idx_v, comb_v, sem_i, sem_p, sem_g):
    wid = lax.axis_index("s") * _NC + lax.axis_index("c")

    @pl.when(wid < _NACT)
    def _():
        base = wid * _BPW
        idx_cp = pltpu.async_copy(idx_hbm.at[pl.ds(base, _BPW)], idx_v, sem_i)
        pos_cp = pltpu.async_copy(
            pos_hbm.at[pl.ds(base, _BPW)], comb_v.at[:, pl.ds(_D, _D)], sem_p
        )
        idx_cp.wait()
        gat = pltpu.async_copy(
            emb_hbm.at[idx_v], comb_v.at[:, pl.ds(0, _D)], sem_g
        )
        pos_cp.wait()
        gat.wait()
        pltpu.sync_copy(comb_v, out_hbm.at[pl.ds(base, _BPW)])


def kernel(fnums, emb_table, pos_table):
    idx = fnums.astype(jnp.int32)
    return _encode(idx, emb_table, pos_table)
